# Initial kernel scaffold; baseline (speedup 1.0000x reference)
import jax
import jax.numpy as jnp
from jax import lax
from jax.experimental import pallas as pl
from jax.experimental.pallas import tpu as pltpu
from jax.experimental.pallas import tpu_sc as plsc

N_EVAL = 32768
N_SEG = 16384
DIM = 3
ROW = 16
L = 16
NC, NS = 2, 16
NW = NC * NS
BPW = N_EVAL // NW
CHUNK = 128
NCHUNK = BPW // CHUNK
BLK = BPW // L


def _seg_and_frac(xe):
    q = (xe * (1.0 / N_SEG)).astype(jnp.int32).astype(jnp.float32)
    t = xe - q * float(N_SEG)
    ti = t.astype(jnp.int32)
    s = t - ti.astype(jnp.float32)
    return jnp.minimum(ti, N_SEG - 1), s


def _bezier_body(xe_hbm, table_hbm, out_hbm, xe_v, idx_v, rows_v, out_v, sem):
    wid = lax.axis_index("s") * NC + lax.axis_index("c")
    base = wid * BPW
    pltpu.sync_copy(xe_hbm.at[pl.ds(base, BPW)], xe_v)

    dmas = []
    for c in range(NCHUNK):
        def seg_ids(b, carry, c=c):
            xe = xe_v[pl.ds(c * CHUNK + b * L, L)]
            ii, _ = _seg_and_frac(xe)
            idx_v[c, pl.ds(b * L, L)] = ii
            return carry

        lax.fori_loop(0, CHUNK // L, seg_ids, 0)
        dmas.append(
            pltpu.async_copy(
                table_hbm.at[idx_v.at[c]],
                rows_v.at[pl.ds(c * CHUNK, CHUNK)],
                sem,
            )
        )
    for dma in dmas:
        dma.wait()

    def blk(b, carry):
        xe = xe_v[pl.ds(b * L, L)]
        _, s = _seg_and_frac(xe)
        omu = 1.0 - s
        s2 = s * s
        o2 = omu * omu
        w = (o2 * omu, 3.0 * s * o2, 3.0 * s2 * omu, s2 * s)
        rid = lax.iota(jnp.int32, L) + b * L
        for d in range(DIM):
            acc = w[0] * plsc.load_gather(
                rows_v, [rid, jnp.full((L,), d, jnp.int32)])
            for j in range(1, 4):
                acc = acc + w[j] * plsc.load_gather(
                    rows_v, [rid, jnp.full((L,), 3 * j + d, jnp.int32)])
            plsc.store_scatter(out_v, [rid, jnp.full((L,), d, jnp.int32)], acc)
        return carry

    lax.fori_loop(0, BLK, blk, 0)
    pltpu.sync_copy(out_v, out_hbm.at[pl.ds(base, BPW)])


def kernel(x_eval, x, control_points):
    del x
    table = jnp.concatenate(
        [control_points.reshape(N_SEG, 4 * DIM),
         jnp.zeros((N_SEG, ROW - 4 * DIM), jnp.float32)], axis=1)
    run = pl.kernel(
        _bezier_body,
        out_type=jax.ShapeDtypeStruct((N_EVAL, DIM), jnp.float32),
        mesh=plsc.VectorSubcoreMesh(core_axis_name="c", subcore_axis_name="s"),
        scratch_types=[
            pltpu.VMEM((BPW,), jnp.float32),
            pltpu.VMEM((NCHUNK, CHUNK), jnp.int32),
            pltpu.VMEM((BPW, ROW), jnp.float32),
            pltpu.VMEM((BPW, DIM), jnp.float32),
            pltpu.SemaphoreType.DMA,
        ],
    )
    return run(x_eval, table)


# speedup vs baseline: 72.2208x; 72.2208x over previous
import jax
import jax.numpy as jnp
from jax import lax
from jax.experimental import pallas as pl
from jax.experimental.pallas import tpu as pltpu
from jax.experimental.pallas import tpu_sc as plsc

N_EVAL = 32768
N_SEG = 16384
DIM = 3
ROW = 16
L = 16
NC, NS = 2, 16
NW = NC * NS
BPW = N_EVAL // NW
CHUNK = 128
NCHUNK = BPW // CHUNK
BLK = BPW // L


def _seg_and_frac(xe):
    q = (xe * (1.0 / N_SEG)).astype(jnp.int32).astype(jnp.float32)
    t = xe - q * float(N_SEG)
    ti = t.astype(jnp.int32)
    s = t - ti.astype(jnp.float32)
    return jnp.minimum(ti, N_SEG - 1), s


def _bezier_body(xe_hbm, table_hbm, out_hbm, xe_v, idx_v, rows_v, out_v, sem):
    wid = lax.axis_index("s") * NC + lax.axis_index("c")
    base = wid * BPW
    pltpu.sync_copy(xe_hbm.at[pl.ds(base, BPW)], xe_v)

    dmas = []
    for c in range(NCHUNK):
        def seg_ids(b, carry, c=c):
            xe = xe_v[pl.ds(c * CHUNK + b * L, L)]
            ii, _ = _seg_and_frac(xe)
            idx_v[c, pl.ds(b * L, L)] = ii
            return carry

        lax.fori_loop(0, CHUNK // L, seg_ids, 0)
        dmas.append(
            pltpu.async_copy(
                table_hbm.at[idx_v.at[c]],
                rows_v.at[pl.ds(c * CHUNK, CHUNK)],
                sem,
            )
        )
    for dma in dmas:
        dma.wait()

    def blk(b, carry):
        xe = xe_v[pl.ds(b * L, L)]
        _, s = _seg_and_frac(xe)
        omu = 1.0 - s
        s2 = s * s
        o2 = omu * omu
        w = (o2 * omu, 3.0 * s * o2, 3.0 * s2 * omu, s2 * s)
        rid = lax.iota(jnp.int32, L) + b * L
        obase = rid * DIM
        for d in range(DIM):
            acc = w[0] * plsc.load_gather(
                rows_v, [rid, jnp.full((L,), d, jnp.int32)])
            for j in range(1, 4):
                acc = acc + w[j] * plsc.load_gather(
                    rows_v, [rid, jnp.full((L,), 3 * j + d, jnp.int32)])
            plsc.store_scatter(out_v, [obase + d], acc)
        return carry

    lax.fori_loop(0, BLK, blk, 0)
    pltpu.sync_copy(out_v, out_hbm.at[pl.ds(base * DIM, BPW * DIM)])


def kernel(x_eval, x, control_points):
    del x
    table = jnp.concatenate(
        [control_points.reshape(N_SEG, 4 * DIM),
         jnp.zeros((N_SEG, ROW - 4 * DIM), jnp.float32)], axis=1)
    run = pl.kernel(
        _bezier_body,
        out_type=jax.ShapeDtypeStruct((N_EVAL * DIM,), jnp.float32),
        mesh=plsc.VectorSubcoreMesh(core_axis_name="c", subcore_axis_name="s"),
        scratch_types=[
            pltpu.VMEM((BPW,), jnp.float32),
            pltpu.VMEM((NCHUNK, CHUNK), jnp.int32),
            pltpu.VMEM((BPW, ROW), jnp.float32),
            pltpu.VMEM((BPW * DIM,), jnp.float32),
            pltpu.SemaphoreType.DMA,
        ],
        compiler_params=pltpu.CompilerParams(
            use_tc_tiling_on_sc=False, needs_layout_passes=False),
    )
    return run(x_eval, table).reshape(N_EVAL, DIM)
